# baseline (speedup 1.0000x reference)
import jax
import jax.numpy as jnp
from jax.experimental import pallas as pl
from jax.experimental.pallas import tpu as pltpu
from jax.experimental.pallas import tpu_sc as plsc

_K = 8192
_C = 256
_NTOK = 8192
_NCHUNK = 2
_CHUNK = _NTOK // _NCHUNK
_TOK_TILE = 256
_K_TILE = 1024
_GATHER_WINDOW = 128


def _argmin_body(zn_ref, cn_ref, z_ref, cb_ref, idx_ref):
    z = z_ref[...]
    zn = zn_ref[...]

    def step(j, carry):
        mins, args = carry
        c = cb_ref[pl.ds(j * _K_TILE, _K_TILE), :]
        t3 = jax.lax.dot_general(
            z, c, (((1,), (1,)), ((), ())),
            preferred_element_type=jnp.float32)
        cn = cn_ref[:, pl.ds(j * _K_TILE, _K_TILE)]
        d = (zn + cn) - 2.0 * t3
        lm = jnp.min(d, axis=1, keepdims=True)
        ii = jax.lax.broadcasted_iota(jnp.int32, d.shape, 1)
        la = jnp.min(jnp.where(d == lm, ii, _K), axis=1, keepdims=True)
        la = la + j * _K_TILE
        upd = lm < mins
        return jnp.where(upd, lm, mins), jnp.where(upd, la, args)

    mins0 = jnp.full((_TOK_TILE, 1), jnp.inf, dtype=jnp.float32)
    args0 = jnp.zeros((_TOK_TILE, 1), dtype=jnp.int32)
    _, args = jax.lax.fori_loop(0, _K // _K_TILE, step, (mins0, args0))
    idx_ref[0, 0, :] = args[:, 0]


def _compute_indices(zn, cn, z_flat, codebook):
    n = z_flat.shape[0]
    idx3 = pl.pallas_call(
        _argmin_body,
        grid=(n // _TOK_TILE,),
        in_specs=[
            pl.BlockSpec((_TOK_TILE, 1), lambda i: (i, 0)),
            pl.BlockSpec((1, _K), lambda i: (0, 0)),
            pl.BlockSpec((_TOK_TILE, _C), lambda i: (i, 0)),
            pl.BlockSpec((_K, _C), lambda i: (0, 0)),
        ],
        out_specs=pl.BlockSpec((1, 1, _TOK_TILE), lambda i: (i, 0, 0)),
        out_shape=jax.ShapeDtypeStruct(
            (n // _TOK_TILE, 1, _TOK_TILE), jnp.int32),
    )(zn, cn, z_flat, codebook)
    return idx3.reshape(n)


def _sc_gather(codebook, indices):
    n = indices.shape[0]
    idx2 = indices.reshape(1, n)
    mesh = plsc.VectorSubcoreMesh(
        core_axis_name="core", subcore_axis_name="subcore")

    @pl.kernel(out_type=jax.ShapeDtypeStruct((n, _C), jnp.float32),
               mesh=mesh)
    def gk(cb_hbm, i_hbm, o_hbm):
        def body(i_vmem, o_vmem):
            pltpu.sync_copy(cb_hbm.at[i_vmem.at[0]], o_vmem)

        pltpu.emit_pipeline(
            body,
            grid=(n // _GATHER_WINDOW,),
            in_specs=[pl.BlockSpec((1, _GATHER_WINDOW),
                                   index_map=lambda i: (0, i))],
            out_specs=[pl.BlockSpec((_GATHER_WINDOW, _C),
                                    index_map=lambda i: (i, 0))],
            core_axis_name=("core", "subcore"),
            dimension_semantics=(pltpu.PARALLEL,),
        )(i_hbm, o_hbm)

    return gk(codebook, idx2)


def _st_loss_body(q_ref, z_ref, qst_ref, acc_ref):
    @pl.when(pl.program_id(0) == 0)
    def _():
        acc_ref[...] = jnp.zeros((1, 1), jnp.float32)
    q = q_ref[...]
    zb = z_ref[...]
    d = q - zb
    qst_ref[...] = zb + d
    acc_ref[...] += jnp.sum(d * d).reshape(1, 1)


def _st_and_loss(q, z_raw):
    n = q.shape[0]
    nblk = 4
    qst, tot = pl.pallas_call(
        _st_loss_body,
        grid=(nblk,),
        in_specs=[
            pl.BlockSpec((n // nblk, _C), lambda i: (i, 0)),
            pl.BlockSpec((n // nblk, _C), lambda i: (i, 0)),
        ],
        out_specs=[
            pl.BlockSpec((n // nblk, _C), lambda i: (i, 0)),
            pl.BlockSpec((1, 1), lambda i: (0, 0)),
        ],
        out_shape=[
            jax.ShapeDtypeStruct((n, _C), jnp.float32),
            jax.ShapeDtypeStruct((1, 1), jnp.float32),
        ],
    )(q, z_raw)
    return qst, tot[0, 0]


def kernel(z, codebook):
    B, C, H, W = z.shape
    z_flat = jnp.transpose(z, (0, 2, 3, 1)).reshape(-1, C)
    zn = jnp.sum(z_flat ** 2, axis=1, keepdims=True)
    cn = jnp.sum(codebook ** 2, axis=1).reshape(1, _K)

    z_raw = z.reshape(_NTOK, C)

    idx_chunks = []
    qst_chunks = []
    partials = []
    for ci in range(_NCHUNK):
        s = ci * _CHUNK
        idx = _compute_indices(zn[s:s + _CHUNK], cn,
                               z_flat[s:s + _CHUNK], codebook)
        q = _sc_gather(codebook, idx)
        qst, part = _st_and_loss(q, z_raw[s:s + _CHUNK])
        idx_chunks.append(idx)
        qst_chunks.append(qst)
        partials.append(part)

    loss = (partials[0] + partials[1]) / jnp.float32(_NTOK * _C)
    indices = jnp.concatenate(idx_chunks)
    quantized_st = jnp.concatenate(qst_chunks).reshape(z.shape)
    return (quantized_st, indices.reshape(B, H, W), loss, loss)


# speedup vs baseline: 1.5407x; 1.5407x over previous
import jax
import jax.numpy as jnp
from jax.experimental import pallas as pl
from jax.experimental.pallas import tpu as pltpu
from jax.experimental.pallas import tpu_sc as plsc

_K = 8192
_C = 256
_NTOK = 8192
_TOK_TILE = 256
_K_TILE = 1024
_GATHER_WINDOW = 128


def _argmin_body(zn_ref, cn_ref, z_ref, cb_ref, idx_ref):
    z = z_ref[...]
    zn = zn_ref[...]
    ii = jax.lax.broadcasted_iota(jnp.int32, (_TOK_TILE, _K_TILE), 1)

    mins = None
    args = None
    for j in range(_K // _K_TILE):
        c = cb_ref[pl.ds(j * _K_TILE, _K_TILE), :]
        t3 = jax.lax.dot_general(
            z, c, (((1,), (1,)), ((), ())),
            preferred_element_type=jnp.float32)
        cn = cn_ref[:, pl.ds(j * _K_TILE, _K_TILE)]
        d = (zn + cn) - 2.0 * t3
        lm = jnp.min(d, axis=1, keepdims=True)
        la = jnp.min(jnp.where(d == lm, ii, _K), axis=1, keepdims=True)
        la = la + j * _K_TILE
        if mins is None:
            mins, args = lm, la
        else:
            upd = lm < mins
            mins = jnp.where(upd, lm, mins)
            args = jnp.where(upd, la, args)
    idx_ref[0, 0, :] = args[:, 0]


def _compute_indices(zn, cn, z_flat, codebook):
    idx3 = pl.pallas_call(
        _argmin_body,
        grid=(_NTOK // _TOK_TILE,),
        in_specs=[
            pl.BlockSpec((_TOK_TILE, 1), lambda i: (i, 0)),
            pl.BlockSpec((1, _K), lambda i: (0, 0)),
            pl.BlockSpec((_TOK_TILE, _C), lambda i: (i, 0)),
            pl.BlockSpec((_K, _C), lambda i: (0, 0)),
        ],
        out_specs=pl.BlockSpec((1, 1, _TOK_TILE), lambda i: (i, 0, 0)),
        out_shape=jax.ShapeDtypeStruct(
            (_NTOK // _TOK_TILE, 1, _TOK_TILE), jnp.int32),
    )(zn, cn, z_flat, codebook)
    return idx3.reshape(_NTOK)


def _sc_gather(codebook, indices):
    idx2 = indices.reshape(1, _NTOK)
    mesh = plsc.VectorSubcoreMesh(
        core_axis_name="core", subcore_axis_name="subcore")

    @pl.kernel(out_type=jax.ShapeDtypeStruct((_NTOK, _C), jnp.float32),
               mesh=mesh)
    def gk(cb_hbm, i_hbm, o_hbm):
        def body(i_vmem, o_vmem):
            pltpu.sync_copy(cb_hbm.at[i_vmem.at[0]], o_vmem)

        pltpu.emit_pipeline(
            body,
            grid=(_NTOK // _GATHER_WINDOW,),
            in_specs=[pl.BlockSpec((1, _GATHER_WINDOW),
                                   index_map=lambda i: (0, i))],
            out_specs=[pl.BlockSpec((_GATHER_WINDOW, _C),
                                    index_map=lambda i: (i, 0))],
            core_axis_name=("core", "subcore"),
            dimension_semantics=(pltpu.PARALLEL,),
        )(i_hbm, o_hbm)

    return gk(codebook, idx2)


def _st_loss_body(q_ref, z_ref, qst_ref, acc_ref):
    @pl.when(pl.program_id(0) == 0)
    def _():
        acc_ref[...] = jnp.zeros((1, 1), jnp.float32)
    q = q_ref[...]
    zb = z_ref[...]
    d = q - zb
    qst_ref[...] = zb + d
    acc_ref[...] += jnp.sum(d * d).reshape(1, 1)


def _st_and_loss(q, z_raw):
    nblk = 8
    qst, tot = pl.pallas_call(
        _st_loss_body,
        grid=(nblk,),
        in_specs=[
            pl.BlockSpec((_NTOK // nblk, _C), lambda i: (i, 0)),
            pl.BlockSpec((_NTOK // nblk, _C), lambda i: (i, 0)),
        ],
        out_specs=[
            pl.BlockSpec((_NTOK // nblk, _C), lambda i: (i, 0)),
            pl.BlockSpec((1, 1), lambda i: (0, 0)),
        ],
        out_shape=[
            jax.ShapeDtypeStruct((_NTOK, _C), jnp.float32),
            jax.ShapeDtypeStruct((1, 1), jnp.float32),
        ],
    )(q, z_raw)
    loss = tot[0, 0] / jnp.float32(_NTOK * _C)
    return qst, loss


def kernel(z, codebook):
    B, C, H, W = z.shape
    z_flat = jnp.transpose(z, (0, 2, 3, 1)).reshape(-1, C)
    zn = jnp.sum(z_flat ** 2, axis=1, keepdims=True)
    cn = jnp.sum(codebook ** 2, axis=1).reshape(1, _K)

    indices = _compute_indices(zn, cn, z_flat, codebook)
    q = _sc_gather(codebook, indices)

    z_raw = z.reshape(_NTOK, C)
    qst, loss = _st_and_loss(q, z_raw)

    quantized_st = qst.reshape(z.shape)
    return (quantized_st, indices.reshape(B, H, W), loss, loss)


# speedup vs baseline: 1.7313x; 1.1237x over previous
import jax
import jax.numpy as jnp
from jax.experimental import pallas as pl
from jax.experimental.pallas import tpu as pltpu
from jax.experimental.pallas import tpu_sc as plsc

_K = 8192
_C = 256
_NTOK = 8192
_TOK_TILE = 256
_K_TILE = 1024
_GATHER_WINDOW = 128


def _argmin_body(zn_ref, cn_ref, z_ref, cb_ref, idx_ref):
    z = z_ref[...]
    zn = zn_ref[...]
    z2 = z * 2.0
    ii = jax.lax.broadcasted_iota(
        jnp.int32, (_TOK_TILE, _K_TILE), 1).astype(jnp.float32)

    mins = None
    args = None
    for j in range(_K // _K_TILE):
        c = cb_ref[pl.ds(j * _K_TILE, _K_TILE), :]
        t3 = jax.lax.dot_general(
            z2, c, (((1,), (1,)), ((), ())),
            preferred_element_type=jnp.float32)
        cn = cn_ref[:, pl.ds(j * _K_TILE, _K_TILE)]
        d = (zn + cn) - t3
        lm = jnp.min(d, axis=1, keepdims=True)
        la = jnp.min(jnp.where(d == lm, ii, jnp.float32(_K)),
                     axis=1, keepdims=True)
        la = la.astype(jnp.int32) + j * _K_TILE
        if mins is None:
            mins, args = lm, la
        else:
            upd = lm < mins
            mins = jnp.where(upd, lm, mins)
            args = jnp.where(upd, la, args)
    idx_ref[0, 0, :] = args[:, 0]


def _compute_indices(zn, cn, z_flat, codebook):
    idx3 = pl.pallas_call(
        _argmin_body,
        grid=(_NTOK // _TOK_TILE,),
        in_specs=[
            pl.BlockSpec((_TOK_TILE, 1), lambda i: (i, 0)),
            pl.BlockSpec((1, _K), lambda i: (0, 0)),
            pl.BlockSpec((_TOK_TILE, _C), lambda i: (i, 0)),
            pl.BlockSpec((_K, _C), lambda i: (0, 0)),
        ],
        out_specs=pl.BlockSpec((1, 1, _TOK_TILE), lambda i: (i, 0, 0)),
        out_shape=jax.ShapeDtypeStruct(
            (_NTOK // _TOK_TILE, 1, _TOK_TILE), jnp.int32),
    )(zn, cn, z_flat, codebook)
    return idx3.reshape(_NTOK)


def _sc_gather(codebook, indices):
    idx2 = indices.reshape(1, _NTOK)
    mesh = plsc.VectorSubcoreMesh(
        core_axis_name="core", subcore_axis_name="subcore")

    @pl.kernel(out_type=jax.ShapeDtypeStruct((_NTOK, _C), jnp.float32),
               mesh=mesh)
    def gk(cb_hbm, i_hbm, o_hbm):
        def body(i_vmem, o_vmem):
            pltpu.sync_copy(cb_hbm.at[i_vmem.at[0]], o_vmem)

        pltpu.emit_pipeline(
            body,
            grid=(_NTOK // _GATHER_WINDOW,),
            in_specs=[pl.BlockSpec((1, _GATHER_WINDOW),
                                   index_map=lambda i: (0, i))],
            out_specs=[pl.BlockSpec((_GATHER_WINDOW, _C),
                                    index_map=lambda i: (i, 0))],
            core_axis_name=("core", "subcore"),
            dimension_semantics=(pltpu.PARALLEL,),
        )(i_hbm, o_hbm)

    return gk(codebook, idx2)


def _st_loss_body(q_ref, z_ref, qst_ref, acc_ref):
    @pl.when(pl.program_id(0) == 0)
    def _():
        acc_ref[...] = jnp.zeros((1, 1), jnp.float32)
    q = q_ref[...]
    zb = z_ref[...]
    d = q - zb
    qst_ref[...] = zb + d
    acc_ref[...] += jnp.sum(d * d).reshape(1, 1)


def _st_and_loss(q, z_raw):
    nblk = 8
    qst, tot = pl.pallas_call(
        _st_loss_body,
        grid=(nblk,),
        in_specs=[
            pl.BlockSpec((_NTOK // nblk, _C), lambda i: (i, 0)),
            pl.BlockSpec((_NTOK // nblk, _C), lambda i: (i, 0)),
        ],
        out_specs=[
            pl.BlockSpec((_NTOK // nblk, _C), lambda i: (i, 0)),
            pl.BlockSpec((1, 1), lambda i: (0, 0)),
        ],
        out_shape=[
            jax.ShapeDtypeStruct((_NTOK, _C), jnp.float32),
            jax.ShapeDtypeStruct((1, 1), jnp.float32),
        ],
    )(q, z_raw)
    loss = tot[0, 0] / jnp.float32(_NTOK * _C)
    return qst, loss


def kernel(z, codebook):
    B, C, H, W = z.shape
    z_flat = jnp.transpose(z, (0, 2, 3, 1)).reshape(-1, C)
    zn = jnp.sum(z_flat ** 2, axis=1, keepdims=True)
    cn = jnp.sum(codebook ** 2, axis=1).reshape(1, _K)

    indices = _compute_indices(zn, cn, z_flat, codebook)
    q = _sc_gather(codebook, indices)

    z_raw = z.reshape(_NTOK, C)
    qst, loss = _st_and_loss(q, z_raw)

    quantized_st = qst.reshape(z.shape)
    return (quantized_st, indices.reshape(B, H, W), loss, loss)
